# baseline (speedup 1.0000x reference)
import functools

import numpy as np
import jax
import jax.numpy as jnp
from jax import lax
from jax.experimental import pallas as pl
from jax.experimental.pallas import tpu as pltpu
from jax.experimental.pallas import tpu_sc as plsc

_VOCAB = 1000000
_EMBED = 64
_BATCH = 4096
_SEQLEN = 200

_NC = 2
_NS = 16
_NW = _NC * _NS

_CHUNK = 128
_NROWS = _BATCH * _SEQLEN
_NCHUNKS = _NROWS // _CHUNK
_CPW = _NCHUNKS // _NW


def _positional_np(seq_len, d_model):
    position = np.arange(seq_len)[:, None].astype(np.float32)
    div_term = np.exp(
        np.arange(0, d_model, 2).astype(np.float32) * -(np.log(10000.0) / d_model)
    )
    pe = np.zeros((seq_len, d_model), dtype=np.float32)
    pe[:, 0::2] = np.sin(position * div_term)
    pe[:, 1::2] = np.cos(position * div_term)
    return pe


def _sc_body(seq_hbm, pe_hbm, table_hbm, out_hbm, idx_v, pe_v, buf_v, gsem):
    wid = lax.axis_index("s") * _NC + lax.axis_index("c")
    base = wid * _CPW

    pltpu.sync_copy(seq_hbm.at[pl.ds(base, _CPW)], idx_v)
    pltpu.sync_copy(pe_hbm, pe_v)

    def chunk_body(c, carry):
        pltpu.async_copy(table_hbm.at[idx_v.at[c]], buf_v, gsem).wait()
        p = lax.rem(c * _CHUNK, _SEQLEN)

        def row_body(r, rcarry):
            for k in range(_EMBED // 16):
                sl = pl.ds(k * 16, 16)
                buf_v[r, sl] = buf_v[r, sl] + pe_v[p + r, sl]
            return rcarry

        lax.fori_loop(0, _CHUNK, row_body, 0, unroll=2)
        pltpu.sync_copy(buf_v, out_hbm.at[base + c])
        return carry

    lax.fori_loop(0, _CPW, chunk_body, 0)


@jax.jit
def kernel(sequence, token_table):
    pe = _positional_np(_SEQLEN, _EMBED)
    pe2 = jnp.asarray(np.concatenate([pe, pe], axis=0))
    seq_flat = sequence.reshape(_NCHUNKS, _CHUNK)

    mesh = plsc.VectorSubcoreMesh(
        core_axis_name="c", subcore_axis_name="s", num_cores=_NC, num_subcores=_NS
    )
    run = pl.kernel(
        _sc_body,
        out_type=jax.ShapeDtypeStruct((_NCHUNKS, _CHUNK, _EMBED), jnp.float32),
        mesh=mesh,
        scratch_types=[
            pltpu.VMEM((_CPW, _CHUNK), jnp.int32),
            pltpu.VMEM((2 * _SEQLEN, _EMBED), jnp.float32),
            pltpu.VMEM((_CHUNK, _EMBED), jnp.float32),
            pltpu.SemaphoreType.DMA,
        ],
        compiler_params=pltpu.CompilerParams(use_tc_tiling_on_sc=False),
    )
    out = run(seq_flat, pe2, token_table)
    return out.reshape(_BATCH, _SEQLEN, _EMBED)


# speedup vs baseline: 1.1254x; 1.1254x over previous
import functools

import numpy as np
import jax
import jax.numpy as jnp
from jax import lax
from jax.experimental import pallas as pl
from jax.experimental.pallas import tpu as pltpu
from jax.experimental.pallas import tpu_sc as plsc

_VOCAB = 1000000
_EMBED = 64
_BATCH = 4096
_SEQLEN = 200

_NC = 2
_NS = 16
_NW = _NC * _NS

_CHUNK = 128
_NROWS = _BATCH * _SEQLEN
_NCHUNKS = _NROWS // _CHUNK
_CPW = _NCHUNKS // _NW


def _positional_np(seq_len, d_model):
    position = np.arange(seq_len)[:, None].astype(np.float32)
    div_term = np.exp(
        np.arange(0, d_model, 2).astype(np.float32) * -(np.log(10000.0) / d_model)
    )
    pe = np.zeros((seq_len, d_model), dtype=np.float32)
    pe[:, 0::2] = np.sin(position * div_term)
    pe[:, 1::2] = np.cos(position * div_term)
    return pe


def _sc_body(
    seq_hbm, pe_hbm, table_hbm, out_hbm,
    idx_v, pe_v, buf0, buf1, gsem0, gsem1, osem0, osem1
):
    wid = lax.axis_index("s") * _NC + lax.axis_index("c")
    base = wid * _CPW

    bufs = (buf0, buf1)
    gsems = (gsem0, gsem1)
    osems = (osem0, osem1)

    pltpu.sync_copy(seq_hbm.at[pl.ds(base, _CPW)], idx_v)
    pltpu.sync_copy(pe_hbm, pe_v)

    def add_pe(c, buf):
        p = lax.rem(c * _CHUNK, _SEQLEN)

        @pl.loop(0, _CHUNK, unroll=4)
        def _row(r):
            for k in range(_EMBED // 16):
                sl = pl.ds(k * 16, 16)
                buf[r, sl] = buf[r, sl] + pe_v[p + r, sl]

    def gather_start(c, b):
        pltpu.async_copy(table_hbm.at[idx_v.at[c]], bufs[b], gsems[b])

    def gather_wait(c, b):
        pltpu.make_async_copy(table_hbm.at[idx_v.at[c]], bufs[b], gsems[b]).wait()

    def out_start(c, b):
        pltpu.async_copy(bufs[b], out_hbm.at[base + c], osems[b])

    def out_wait(c, b):
        pltpu.make_async_copy(bufs[b], out_hbm.at[base + c], osems[b]).wait()

    gather_start(0, 0)
    gather_wait(0, 0)
    gather_start(1, 1)
    add_pe(0, buf0)
    out_start(0, 0)

    @pl.loop(1, _CPW - 1, step=2)
    def _steady(g):
        for b, off in ((1, 0), (0, 1)):
            c = g + off
            nb = 1 - b
            gather_wait(c, b)
            out_wait(c - 1, nb)
            gather_start(c + 1, nb)
            add_pe(c, bufs[b])
            out_start(c, b)

    last = _CPW - 1
    gather_wait(last, 1)
    out_wait(last - 1, 0)
    add_pe(last, buf1)
    pltpu.sync_copy(buf1, out_hbm.at[base + last])


@jax.jit
def kernel(sequence, token_table):
    pe = _positional_np(_SEQLEN, _EMBED)
    pe2 = jnp.asarray(np.concatenate([pe, pe], axis=0))
    seq_flat = sequence.reshape(_NCHUNKS, _CHUNK)

    mesh = plsc.VectorSubcoreMesh(
        core_axis_name="c", subcore_axis_name="s", num_cores=_NC, num_subcores=_NS
    )
    run = pl.kernel(
        _sc_body,
        out_type=jax.ShapeDtypeStruct((_NCHUNKS, _CHUNK, _EMBED), jnp.float32),
        mesh=mesh,
        scratch_types=[
            pltpu.VMEM((_CPW, _CHUNK), jnp.int32),
            pltpu.VMEM((2 * _SEQLEN, _EMBED), jnp.float32),
            pltpu.VMEM((_CHUNK, _EMBED), jnp.float32),
            pltpu.VMEM((_CHUNK, _EMBED), jnp.float32),
            pltpu.SemaphoreType.DMA,
            pltpu.SemaphoreType.DMA,
            pltpu.SemaphoreType.DMA,
            pltpu.SemaphoreType.DMA,
        ],
        compiler_params=pltpu.CompilerParams(use_tc_tiling_on_sc=False),
    )
    out = run(seq_flat, pe2, token_table)
    return out.reshape(_BATCH, _SEQLEN, _EMBED)


# speedup vs baseline: 1.4851x; 1.3196x over previous
import functools

import numpy as np
import jax
import jax.numpy as jnp
from jax import lax
from jax.experimental import pallas as pl
from jax.experimental.pallas import tpu as pltpu
from jax.experimental.pallas import tpu_sc as plsc

_VOCAB = 1000000
_EMBED = 64
_BATCH = 4096
_SEQLEN = 200

_NC = 2
_NS = 16
_NW = _NC * _NS

_CHUNK = 128
_NROWS = _BATCH * _SEQLEN
_NCHUNKS = _NROWS // _CHUNK
_CPW = _NCHUNKS // _NW


def _positional_np(seq_len, d_model):
    position = np.arange(seq_len)[:, None].astype(np.float32)
    div_term = np.exp(
        np.arange(0, d_model, 2).astype(np.float32) * -(np.log(10000.0) / d_model)
    )
    pe = np.zeros((seq_len, d_model), dtype=np.float32)
    pe[:, 0::2] = np.sin(position * div_term)
    pe[:, 1::2] = np.cos(position * div_term)
    return pe


_NBUF = 5


def _sc_body(seq_hbm, pe_hbm, table_hbm, out_hbm, idx_v, pe_v, bufs, gsems, osems):
    wid = lax.axis_index("s") * _NC + lax.axis_index("c")
    base = wid * _CPW

    pltpu.sync_copy(seq_hbm.at[pl.ds(base, _CPW)], idx_v)
    pltpu.sync_copy(pe_hbm, pe_v)

    def add_pe(c, buf):
        p = lax.rem(c * _CHUNK, _SEQLEN)

        @functools.partial(plsc.parallel_loop, 0, _CHUNK, unroll=4)
        def _row(r):
            for k in range(_EMBED // 16):
                sl = pl.ds(k * 16, 16)
                buf[r, sl] = buf[r, sl] + pe_v[p + r, sl]

    def gather_start(c, b):
        pltpu.async_copy(table_hbm.at[idx_v.at[c]], bufs[b], gsems[b])

    def gather_wait(c, b):
        pltpu.make_async_copy(table_hbm.at[idx_v.at[c]], bufs[b], gsems[b]).wait()

    def out_start(c, b):
        pltpu.async_copy(bufs[b], out_hbm.at[base + c], osems[b])

    def out_wait(c, b):
        pltpu.make_async_copy(bufs[b], out_hbm.at[base + c], osems[b]).wait()

    for c in range(_NBUF - 1):
        gather_start(c, c)

    @pl.loop(0, _CPW, step=_NBUF)
    def _steady(g):
        for j in range(_NBUF):
            c = g + j
            nb = (j + _NBUF - 1) % _NBUF
            gather_wait(c, j)

            @pl.when(c + _NBUF - 1 < _CPW)
            def _prefetch(c=c, nb=nb):
                @pl.when(c >= 1)
                def _drain_prev():
                    out_wait(c - 1, nb)

                gather_start(c + _NBUF - 1, nb)

            add_pe(c, bufs[j])
            out_start(c, j)

    for c in range(_CPW - _NBUF, _CPW):
        out_wait(c, c % _NBUF)


@jax.jit
def kernel(sequence, token_table):
    pe = _positional_np(_SEQLEN, _EMBED)
    pe2 = jnp.asarray(np.concatenate([pe, pe], axis=0))
    seq_flat = sequence.reshape(_NCHUNKS, _CHUNK)

    mesh = plsc.VectorSubcoreMesh(
        core_axis_name="c", subcore_axis_name="s", num_cores=_NC, num_subcores=_NS
    )
    run = pl.kernel(
        _sc_body,
        out_type=jax.ShapeDtypeStruct((_NCHUNKS, _CHUNK, _EMBED), jnp.float32),
        mesh=mesh,
        scratch_types=[
            pltpu.VMEM((_CPW, _CHUNK), jnp.int32),
            pltpu.VMEM((2 * _SEQLEN, _EMBED), jnp.float32),
            tuple(pltpu.VMEM((_CHUNK, _EMBED), jnp.float32) for _ in range(_NBUF)),
            tuple(pltpu.SemaphoreType.DMA for _ in range(_NBUF)),
            tuple(pltpu.SemaphoreType.DMA for _ in range(_NBUF)),
        ],
        compiler_params=pltpu.CompilerParams(use_tc_tiling_on_sc=False),
    )
    out = run(seq_flat, pe2, token_table)
    return out.reshape(_BATCH, _SEQLEN, _EMBED)
